# baseline (speedup 1.0000x reference)
import functools

import jax
import jax.numpy as jnp
from jax import lax
from jax.experimental import pallas as pl
from jax.experimental.pallas import tpu as pltpu
from jax.experimental.pallas import tpu_sc as plsc

N = 10000
E = 320000
NC, NS = 2, 16
K = 128
ACC_ROWS = 10240
RPT = 624
TAIL = N - NS * RPT
NBUF = 8
DEG_W = 16

CH_COL = 160
CH_EDGE = 80

BR = 10000
GRID = N // BR

_SC_PARAMS = pltpu.CompilerParams(use_tc_tiling_on_sc=False)


def _sc_gather_scatter(D, split_cols):
    D2 = D // 2 if split_cols else D
    CH = CH_COL if split_cols else CH_EDGE
    NG = CH // NBUF
    mesh = plsc.VectorSubcoreMesh(core_axis_name="c", subcore_axis_name="s")
    out_shape = (N, D) if split_cols else (NC, N, D)

    @functools.partial(
        pl.kernel,
        out_type=jax.ShapeDtypeStruct(out_shape, jnp.float32),
        mesh=mesh,
        scratch_types=[
            pltpu.VMEM((CH, K), jnp.int32),
            pltpu.VMEM((CH, K), jnp.int32),
            [pltpu.VMEM((K, D2), jnp.float32)] * NBUF,
            pltpu.VMEM_SHARED((N, D2), jnp.float32),
            pltpu.VMEM_SHARED((ACC_ROWS, D2), jnp.float32),
            [pltpu.SemaphoreType.DMA] * NBUF,
        ],
        compiler_params=_SC_PARAMS,
    )
    def k(hs, src_i, dst_i, out, src_v, dst_v, rows, hs_s, acc, sem):
        c = lax.axis_index("c")
        s = lax.axis_index("s")
        if split_cols:
            pltpu.sync_copy(src_i.at[s], src_v)
            pltpu.sync_copy(dst_i.at[s], dst_v)
        else:
            pltpu.sync_copy(src_i.at[s, pl.ds(c * CH, CH)], src_v)
            pltpu.sync_copy(dst_i.at[s, pl.ds(c * CH, CH)], dst_v)
        base = s * RPT
        co = c * D2 if split_cols else 0

        def stage(r0, nr):
            sl = (pl.ds(r0, nr), pl.ds(co, D2)) if split_cols else (pl.ds(r0, nr),)
            pltpu.sync_copy(hs.at[sl], hs_s.at[pl.ds(r0, nr)])
            pltpu.sync_copy(hs.at[sl], acc.at[pl.ds(r0, nr)])

        stage(base, RPT)

        @pl.when(s == NS - 1)
        def _():
            stage(NS * RPT, TAIL)

        plsc.subcore_barrier()

        for b in range(NBUF):
            pltpu.async_copy(hs_s.at[src_v.at[b]], rows[b], sem[b])

        def group(g, carry):
            for b in range(NBUF):
                i = g * NBUF + b
                pltpu.make_async_copy(hs_s.at[src_v.at[i]], rows[b],
                                      sem[b]).wait()
                pltpu.async_copy(rows[b], acc.at[dst_v.at[i]], sem[b],
                                 add=True)
            for b in range(NBUF):
                i = g * NBUF + b
                pltpu.make_async_copy(rows[b], acc.at[dst_v.at[i]],
                                      sem[b]).wait()

                @pl.when(g < NG - 1)
                def _():
                    j = (g + 1) * NBUF + b
                    pltpu.async_copy(hs_s.at[src_v.at[j]], rows[b], sem[b])

            return carry

        lax.fori_loop(0, NG, group, 0)
        plsc.subcore_barrier()

        def emit(r0, nr):
            sl = ((pl.ds(r0, nr), pl.ds(co, D2)) if split_cols
                  else (c, pl.ds(r0, nr)))
            pltpu.sync_copy(acc.at[pl.ds(r0, nr)], out.at[sl])

        emit(base, RPT)

        @pl.when(s == NS - 1)
        def _():
            emit(NS * RPT, TAIL)

    return k


def _sc_degree():
    mesh = plsc.VectorSubcoreMesh(core_axis_name="c", subcore_axis_name="s")
    rpt_deg = ACC_ROWS // NS
    NG = CH_EDGE // NBUF

    @functools.partial(
        pl.kernel,
        out_type=jax.ShapeDtypeStruct((NC, N, DEG_W), jnp.float32),
        mesh=mesh,
        scratch_types=[
            pltpu.VMEM((CH_EDGE, K), jnp.int32),
            pltpu.VMEM((K, DEG_W), jnp.float32),
            pltpu.VMEM_SHARED((ACC_ROWS, DEG_W), jnp.float32),
            [pltpu.SemaphoreType.DMA] * NBUF,
        ],
        compiler_params=_SC_PARAMS,
    )
    def k(ones, dst_i, out, dst_v, rows, acc, sem):
        c = lax.axis_index("c")
        s = lax.axis_index("s")
        pltpu.sync_copy(dst_i.at[s, pl.ds(c * CH_EDGE, CH_EDGE)], dst_v)
        pltpu.sync_copy(ones, rows)
        for j in range(rpt_deg // K):
            pltpu.sync_copy(rows, acc.at[pl.ds(s * rpt_deg + j * K, K)])
        plsc.subcore_barrier()

        def body(g, carry):
            for b in range(NBUF):
                i = g * NBUF + b
                pltpu.async_copy(rows, acc.at[dst_v.at[i]], sem[b], add=True)
            for b in range(NBUF):
                i = g * NBUF + b
                pltpu.make_async_copy(rows, acc.at[dst_v.at[i]],
                                      sem[b]).wait()
            return carry

        lax.fori_loop(0, NG, body, 0)
        plsc.subcore_barrier()
        base = s * RPT
        pltpu.sync_copy(acc.at[pl.ds(base, RPT)], out.at[c, pl.ds(base, RPT)])

        @pl.when(s == NS - 1)
        def _():
            pltpu.sync_copy(acc.at[pl.ds(NS * RPT, TAIL)],
                            out.at[c, pl.ds(NS * RPT, TAIL)])

    return k


def _dinv(d):
    return lax.rsqrt(d[0, :, :1] + d[1, :, :1] - 1.0)


def _row_spec(d):
    return pl.BlockSpec((BR, d), lambda i: (i, 0))


def _row3_spec(d):
    return pl.BlockSpec((NC, BR, d), lambda i: (0, i, 0))


def _full_spec(r, c):
    return pl.BlockSpec((r, c), lambda i: (0, 0))


def _mm1(x, w, degp):
    def body(x_r, w_r, d_r, o_r):
        h = jnp.dot(x_r[...], w_r[...], preferred_element_type=jnp.float32)
        o_r[...] = h * _dinv(d_r[...])

    return pl.pallas_call(
        body,
        grid=(GRID,),
        in_specs=[_row_spec(128), _full_spec(128, 64), _row3_spec(DEG_W)],
        out_specs=_row_spec(64),
        out_shape=jax.ShapeDtypeStruct((N, 64), jnp.float32),
    )(x, w, degp)


def _mm_mid(p, degp, w, b, din, dout, hs=None):
    pair = p.ndim == 3

    def body(*refs):
        if pair:
            p_r, hs_r, d_r, w_r, b_r, f_r, h_r = refs
            agg = p_r[0] + p_r[1] - hs_r[...]
        else:
            p_r, d_r, w_r, b_r, f_r, h_r = refs
            agg = p_r[...]
        dinv = _dinv(d_r[...])
        f = jnp.maximum(dinv * agg + b_r[...], 0.0)
        f_r[...] = f
        h_r[...] = jnp.dot(f, w_r[...], preferred_element_type=jnp.float32) * dinv

    if pair:
        args = [p, hs]
        specs = [_row3_spec(din), _row_spec(din)]
    else:
        args = [p]
        specs = [_row_spec(din)]
    return pl.pallas_call(
        body,
        grid=(GRID,),
        in_specs=specs + [_row3_spec(DEG_W),
                          _full_spec(din, dout), _full_spec(1, din)],
        out_specs=[_row_spec(din), _row_spec(dout)],
        out_shape=[jax.ShapeDtypeStruct((N, din), jnp.float32),
                   jax.ShapeDtypeStruct((N, dout), jnp.float32)],
    )(*args, degp, w, b)


def _mm_fc12(f1, f2, wf1, wf2, bfc):
    def body(f1_r, f2_r, w1_r, w2_r, bf_r, o_r):
        o_r[...] = (jnp.dot(f1_r[...], w1_r[...],
                            preferred_element_type=jnp.float32)
                    + jnp.dot(f2_r[...], w2_r[...],
                              preferred_element_type=jnp.float32)
                    + bf_r[...])

    return pl.pallas_call(
        body,
        grid=(GRID,),
        in_specs=[_row_spec(64), _row_spec(32),
                  _full_spec(64, 16), _full_spec(32, 16), _full_spec(1, 16)],
        out_specs=_row_spec(16),
        out_shape=jax.ShapeDtypeStruct((N, 16), jnp.float32),
    )(f1, f2, wf1, wf2, bfc)


def _mm_fin(p3, h3s, degp, fc12, b3, wf3):
    def body(p_r, hs_r, d_r, fc_r, b3_r, w3_r, o_r):
        dinv = _dinv(d_r[...])
        agg = p_r[0] + p_r[1] - hs_r[...]
        f3 = jnp.maximum(dinv * agg + b3_r[...], 0.0)
        acc = fc_r[...] + jnp.dot(f3, w3_r[...],
                                  preferred_element_type=jnp.float32)
        o_r[...] = jnp.maximum(acc, 0.0)

    return pl.pallas_call(
        body,
        grid=(GRID,),
        in_specs=[_row3_spec(16), _row_spec(16), _row3_spec(DEG_W),
                  _row_spec(16), _full_spec(1, 16), _full_spec(16, 16)],
        out_specs=_row_spec(16),
        out_shape=jax.ShapeDtypeStruct((N, 16), jnp.float32),
    )(p3, h3s, degp, fc12, b3, wf3)


def _pad_spread(idx, nper, npad, dst):
    if dst:
        pad = N + (jnp.arange(npad, dtype=jnp.int32) % (ACC_ROWS - N))
    else:
        pad = (jnp.arange(npad, dtype=jnp.int32) * 97) % N
    lead = idx.reshape(-1, nper)
    return jnp.concatenate(
        [lead, jnp.broadcast_to(pad, (lead.shape[0], npad))], axis=1)


def kernel(edges, features, W1, b1, W2, b2, W3, b3, Wfc, bfc):
    src = edges[0].astype(jnp.int32)
    dst = edges[1].astype(jnp.int32)
    ept = E // NS
    npad = CH_COL * K - ept
    src_c = _pad_spread(src, ept, npad, False).reshape(NS, CH_COL, K)
    dst_c = _pad_spread(dst, ept, npad, True).reshape(NS, CH_COL, K)
    ones = jnp.ones((K, DEG_W), jnp.float32)

    degp = _sc_degree()(ones, dst_c)

    h1s = _mm1(features, W1, degp)
    p1 = _sc_gather_scatter(64, split_cols=True)(h1s, src_c, dst_c)
    f1, h2s = _mm_mid(p1, degp, W2, b1.reshape(1, 64), 64, 32)
    p2 = _sc_gather_scatter(32, split_cols=False)(h2s, src_c, dst_c)
    f2, h3s = _mm_mid(p2, degp, W3, b2.reshape(1, 32), 32, 16, hs=h2s)
    p3 = _sc_gather_scatter(16, split_cols=False)(h3s, src_c, dst_c)
    fc12 = _mm_fc12(f1, f2, Wfc[:64], Wfc[64:96], bfc.reshape(1, 16))
    return _mm_fin(p3, h3s, degp, fc12, b3.reshape(1, 16), Wfc[96:])


# speedup vs baseline: 1.0113x; 1.0113x over previous
import functools

import jax
import jax.numpy as jnp
from jax import lax
from jax.experimental import pallas as pl
from jax.experimental.pallas import tpu as pltpu
from jax.experimental.pallas import tpu_sc as plsc

N = 10000
E = 320000
NC, NS = 2, 16
K = 128
ACC_ROWS = 10240
RPT = 624
TAIL = N - NS * RPT
NBUF = 8
DEG_W = 16

CH_COL = 160
CH_EDGE = 80

BR = 5000
GRID = N // BR

_SC_PARAMS = pltpu.CompilerParams(use_tc_tiling_on_sc=False)


def _sc_gather_scatter(D, split_cols):
    D2 = D // 2 if split_cols else D
    CH = CH_COL if split_cols else CH_EDGE
    NG = CH // NBUF
    mesh = plsc.VectorSubcoreMesh(core_axis_name="c", subcore_axis_name="s")
    out_shape = (N, D) if split_cols else (NC, N, D)

    @functools.partial(
        pl.kernel,
        out_type=jax.ShapeDtypeStruct(out_shape, jnp.float32),
        mesh=mesh,
        scratch_types=[
            pltpu.VMEM((CH, K), jnp.int32),
            pltpu.VMEM((CH, K), jnp.int32),
            [pltpu.VMEM((K, D2), jnp.float32)] * NBUF,
            pltpu.VMEM_SHARED((N, D2), jnp.float32),
            pltpu.VMEM_SHARED((ACC_ROWS, D2), jnp.float32),
            [pltpu.SemaphoreType.DMA] * NBUF,
        ],
        compiler_params=_SC_PARAMS,
    )
    def k(hs, src_i, dst_i, out, src_v, dst_v, rows, hs_s, acc, sem):
        c = lax.axis_index("c")
        s = lax.axis_index("s")
        if split_cols:
            pltpu.sync_copy(src_i.at[s], src_v)
            pltpu.sync_copy(dst_i.at[s], dst_v)
        else:
            pltpu.sync_copy(src_i.at[s, pl.ds(c * CH, CH)], src_v)
            pltpu.sync_copy(dst_i.at[s, pl.ds(c * CH, CH)], dst_v)
        base = s * RPT
        co = c * D2 if split_cols else 0

        def stage(r0, nr):
            sl = (pl.ds(r0, nr), pl.ds(co, D2)) if split_cols else (pl.ds(r0, nr),)
            pltpu.sync_copy(hs.at[sl], hs_s.at[pl.ds(r0, nr)])
            pltpu.sync_copy(hs.at[sl], acc.at[pl.ds(r0, nr)])

        stage(base, RPT)

        @pl.when(s == NS - 1)
        def _():
            stage(NS * RPT, TAIL)

        plsc.subcore_barrier()

        for b in range(NBUF):
            pltpu.async_copy(hs_s.at[src_v.at[b]], rows[b], sem[b])

        def group(g, carry):
            for b in range(NBUF):
                i = g * NBUF + b
                pltpu.make_async_copy(hs_s.at[src_v.at[i]], rows[b],
                                      sem[b]).wait()
                pltpu.async_copy(rows[b], acc.at[dst_v.at[i]], sem[b],
                                 add=True)
            for b in range(NBUF):
                i = g * NBUF + b
                pltpu.make_async_copy(rows[b], acc.at[dst_v.at[i]],
                                      sem[b]).wait()

                @pl.when(g < NG - 1)
                def _():
                    j = (g + 1) * NBUF + b
                    pltpu.async_copy(hs_s.at[src_v.at[j]], rows[b], sem[b])

            return carry

        lax.fori_loop(0, NG, group, 0)
        plsc.subcore_barrier()

        def emit(r0, nr):
            sl = ((pl.ds(r0, nr), pl.ds(co, D2)) if split_cols
                  else (c, pl.ds(r0, nr)))
            pltpu.sync_copy(acc.at[pl.ds(r0, nr)], out.at[sl])

        emit(base, RPT)

        @pl.when(s == NS - 1)
        def _():
            emit(NS * RPT, TAIL)

    return k


def _sc_degree():
    mesh = plsc.VectorSubcoreMesh(core_axis_name="c", subcore_axis_name="s")
    rpt_deg = ACC_ROWS // NS
    NG = CH_EDGE // NBUF

    @functools.partial(
        pl.kernel,
        out_type=jax.ShapeDtypeStruct((NC, N, DEG_W), jnp.float32),
        mesh=mesh,
        scratch_types=[
            pltpu.VMEM((CH_EDGE, K), jnp.int32),
            pltpu.VMEM((K, DEG_W), jnp.float32),
            pltpu.VMEM_SHARED((ACC_ROWS, DEG_W), jnp.float32),
            [pltpu.SemaphoreType.DMA] * NBUF,
        ],
        compiler_params=_SC_PARAMS,
    )
    def k(ones, dst_i, out, dst_v, rows, acc, sem):
        c = lax.axis_index("c")
        s = lax.axis_index("s")
        pltpu.sync_copy(dst_i.at[s, pl.ds(c * CH_EDGE, CH_EDGE)], dst_v)
        pltpu.sync_copy(ones, rows)
        for j in range(rpt_deg // K):
            pltpu.sync_copy(rows, acc.at[pl.ds(s * rpt_deg + j * K, K)])
        plsc.subcore_barrier()

        def body(g, carry):
            for b in range(NBUF):
                i = g * NBUF + b
                pltpu.async_copy(rows, acc.at[dst_v.at[i]], sem[b], add=True)
            for b in range(NBUF):
                i = g * NBUF + b
                pltpu.make_async_copy(rows, acc.at[dst_v.at[i]],
                                      sem[b]).wait()
            return carry

        lax.fori_loop(0, NG, body, 0)
        plsc.subcore_barrier()
        base = s * RPT
        pltpu.sync_copy(acc.at[pl.ds(base, RPT)], out.at[c, pl.ds(base, RPT)])

        @pl.when(s == NS - 1)
        def _():
            pltpu.sync_copy(acc.at[pl.ds(NS * RPT, TAIL)],
                            out.at[c, pl.ds(NS * RPT, TAIL)])

    return k


def _dinv(d):
    return lax.rsqrt(d[0, :, :1] + d[1, :, :1] - 1.0)


def _row_spec(d):
    return pl.BlockSpec((BR, d), lambda i: (i, 0))


def _row3_spec(d):
    return pl.BlockSpec((NC, BR, d), lambda i: (0, i, 0))


def _full_spec(r, c):
    return pl.BlockSpec((r, c), lambda i: (0, 0))


def _mm1(x, w, degp):
    def body(x_r, w_r, d_r, o_r):
        h = jnp.dot(x_r[...], w_r[...], preferred_element_type=jnp.float32)
        o_r[...] = h * _dinv(d_r[...])

    return pl.pallas_call(
        body,
        grid=(GRID,),
        in_specs=[_row_spec(128), _full_spec(128, 64), _row3_spec(DEG_W)],
        out_specs=_row_spec(64),
        out_shape=jax.ShapeDtypeStruct((N, 64), jnp.float32),
    )(x, w, degp)


def _mm_mid(p, degp, w, b, din, dout, hs=None):
    pair = p.ndim == 3

    def body(*refs):
        if pair:
            p_r, hs_r, d_r, w_r, b_r, f_r, h_r = refs
            agg = p_r[0] + p_r[1] - hs_r[...]
        else:
            p_r, d_r, w_r, b_r, f_r, h_r = refs
            agg = p_r[...]
        dinv = _dinv(d_r[...])
        f = jnp.maximum(dinv * agg + b_r[...], 0.0)
        f_r[...] = f
        h_r[...] = jnp.dot(f, w_r[...], preferred_element_type=jnp.float32) * dinv

    if pair:
        args = [p, hs]
        specs = [_row3_spec(din), _row_spec(din)]
    else:
        args = [p]
        specs = [_row_spec(din)]
    return pl.pallas_call(
        body,
        grid=(GRID,),
        in_specs=specs + [_row3_spec(DEG_W),
                          _full_spec(din, dout), _full_spec(1, din)],
        out_specs=[_row_spec(din), _row_spec(dout)],
        out_shape=[jax.ShapeDtypeStruct((N, din), jnp.float32),
                   jax.ShapeDtypeStruct((N, dout), jnp.float32)],
    )(*args, degp, w, b)


def _mm_fc12(f1, f2, wf1, wf2, bfc):
    def body(f1_r, f2_r, w1_r, w2_r, bf_r, o_r):
        o_r[...] = (jnp.dot(f1_r[...], w1_r[...],
                            preferred_element_type=jnp.float32)
                    + jnp.dot(f2_r[...], w2_r[...],
                              preferred_element_type=jnp.float32)
                    + bf_r[...])

    return pl.pallas_call(
        body,
        grid=(GRID,),
        in_specs=[_row_spec(64), _row_spec(32),
                  _full_spec(64, 16), _full_spec(32, 16), _full_spec(1, 16)],
        out_specs=_row_spec(16),
        out_shape=jax.ShapeDtypeStruct((N, 16), jnp.float32),
    )(f1, f2, wf1, wf2, bfc)


def _mm_fin(p3, h3s, degp, fc12, b3, wf3):
    def body(p_r, hs_r, d_r, fc_r, b3_r, w3_r, o_r):
        dinv = _dinv(d_r[...])
        agg = p_r[0] + p_r[1] - hs_r[...]
        f3 = jnp.maximum(dinv * agg + b3_r[...], 0.0)
        acc = fc_r[...] + jnp.dot(f3, w3_r[...],
                                  preferred_element_type=jnp.float32)
        o_r[...] = jnp.maximum(acc, 0.0)

    return pl.pallas_call(
        body,
        grid=(GRID,),
        in_specs=[_row3_spec(16), _row_spec(16), _row3_spec(DEG_W),
                  _row_spec(16), _full_spec(1, 16), _full_spec(16, 16)],
        out_specs=_row_spec(16),
        out_shape=jax.ShapeDtypeStruct((N, 16), jnp.float32),
    )(p3, h3s, degp, fc12, b3, wf3)


def _pad_spread(idx, nper, npad, dst):
    if dst:
        pad = N + (jnp.arange(npad, dtype=jnp.int32) % (ACC_ROWS - N))
    else:
        pad = (jnp.arange(npad, dtype=jnp.int32) * 97) % N
    lead = idx.reshape(-1, nper)
    return jnp.concatenate(
        [lead, jnp.broadcast_to(pad, (lead.shape[0], npad))], axis=1)


def kernel(edges, features, W1, b1, W2, b2, W3, b3, Wfc, bfc):
    src = edges[0].astype(jnp.int32)
    dst = edges[1].astype(jnp.int32)
    ept = E // NS
    npad = CH_COL * K - ept
    src_c = _pad_spread(src, ept, npad, False).reshape(NS, CH_COL, K)
    dst_c = _pad_spread(dst, ept, npad, True).reshape(NS, CH_COL, K)
    ones = jnp.ones((K, DEG_W), jnp.float32)

    degp = _sc_degree()(ones, dst_c)

    h1s = _mm1(features, W1, degp)
    p1 = _sc_gather_scatter(64, split_cols=True)(h1s, src_c, dst_c)
    f1, h2s = _mm_mid(p1, degp, W2, b1.reshape(1, 64), 64, 32)
    p2 = _sc_gather_scatter(32, split_cols=False)(h2s, src_c, dst_c)
    f2, h3s = _mm_mid(p2, degp, W3, b2.reshape(1, 32), 32, 16, hs=h2s)
    p3 = _sc_gather_scatter(16, split_cols=False)(h3s, src_c, dst_c)
    fc12 = _mm_fc12(f1, f2, Wfc[:64], Wfc[64:96], bfc.reshape(1, 16))
    return _mm_fin(p3, h3s, degp, fc12, b3.reshape(1, 16), Wfc[96:])


# speedup vs baseline: 1.0309x; 1.0194x over previous
import functools

import jax
import jax.numpy as jnp
from jax import lax
from jax.experimental import pallas as pl
from jax.experimental.pallas import tpu as pltpu
from jax.experimental.pallas import tpu_sc as plsc

N = 10000
E = 320000
NC, NS = 2, 16
K = 128
ACC_ROWS = 10240
RPT = 624
TAIL = N - NS * RPT
NBUF = 10
DEG_W = 16

CH_COL = 160
CH_EDGE = 80

BR = 5000
GRID = N // BR

_SC_PARAMS = pltpu.CompilerParams(use_tc_tiling_on_sc=False)


def _sc_gather_scatter(D, split_cols):
    D2 = D // 2 if split_cols else D
    CH = CH_COL if split_cols else CH_EDGE
    NG = CH // NBUF
    mesh = plsc.VectorSubcoreMesh(core_axis_name="c", subcore_axis_name="s")
    out_shape = (N, D) if split_cols else (NC, N, D)

    @functools.partial(
        pl.kernel,
        out_type=jax.ShapeDtypeStruct(out_shape, jnp.float32),
        mesh=mesh,
        scratch_types=[
            pltpu.VMEM((CH, K), jnp.int32),
            pltpu.VMEM((CH, K), jnp.int32),
            [pltpu.VMEM((K, D2), jnp.float32)] * NBUF,
            pltpu.VMEM_SHARED((N, D2), jnp.float32),
            pltpu.VMEM_SHARED((ACC_ROWS, D2), jnp.float32),
            [pltpu.SemaphoreType.DMA] * NBUF,
        ],
        compiler_params=_SC_PARAMS,
    )
    def k(hs, src_i, dst_i, out, src_v, dst_v, rows, hs_s, acc, sem):
        c = lax.axis_index("c")
        s = lax.axis_index("s")
        if split_cols:
            pltpu.sync_copy(src_i.at[s], src_v)
            pltpu.sync_copy(dst_i.at[s], dst_v)
        else:
            pltpu.sync_copy(src_i.at[s, pl.ds(c * CH, CH)], src_v)
            pltpu.sync_copy(dst_i.at[s, pl.ds(c * CH, CH)], dst_v)
        base = s * RPT
        co = c * D2 if split_cols else 0

        def stage(r0, nr):
            sl = (pl.ds(r0, nr), pl.ds(co, D2)) if split_cols else (pl.ds(r0, nr),)
            pltpu.sync_copy(hs.at[sl], hs_s.at[pl.ds(r0, nr)])
            pltpu.sync_copy(hs.at[sl], acc.at[pl.ds(r0, nr)])

        stage(base, RPT)

        @pl.when(s == NS - 1)
        def _():
            stage(NS * RPT, TAIL)

        plsc.subcore_barrier()

        for b in range(NBUF):
            pltpu.async_copy(hs_s.at[src_v.at[b]], rows[b], sem[b])

        def group(g, carry):
            for b in range(NBUF):
                i = g * NBUF + b
                pltpu.make_async_copy(hs_s.at[src_v.at[i]], rows[b],
                                      sem[b]).wait()
                pltpu.async_copy(rows[b], acc.at[dst_v.at[i]], sem[b],
                                 add=True)
            for b in range(NBUF):
                i = g * NBUF + b
                pltpu.make_async_copy(rows[b], acc.at[dst_v.at[i]],
                                      sem[b]).wait()

                @pl.when(g < NG - 1)
                def _():
                    j = (g + 1) * NBUF + b
                    pltpu.async_copy(hs_s.at[src_v.at[j]], rows[b], sem[b])

            return carry

        lax.fori_loop(0, NG, group, 0)
        plsc.subcore_barrier()

        def emit(r0, nr):
            sl = ((pl.ds(r0, nr), pl.ds(co, D2)) if split_cols
                  else (c, pl.ds(r0, nr)))
            pltpu.sync_copy(acc.at[pl.ds(r0, nr)], out.at[sl])

        emit(base, RPT)

        @pl.when(s == NS - 1)
        def _():
            emit(NS * RPT, TAIL)

    return k


def _sc_degree():
    mesh = plsc.VectorSubcoreMesh(core_axis_name="c", subcore_axis_name="s")
    rpt_deg = ACC_ROWS // NS
    NG = CH_EDGE // NBUF

    @functools.partial(
        pl.kernel,
        out_type=jax.ShapeDtypeStruct((NC, N, DEG_W), jnp.float32),
        mesh=mesh,
        scratch_types=[
            pltpu.VMEM((CH_EDGE, K), jnp.int32),
            pltpu.VMEM((K, DEG_W), jnp.float32),
            pltpu.VMEM_SHARED((ACC_ROWS, DEG_W), jnp.float32),
            [pltpu.SemaphoreType.DMA] * NBUF,
        ],
        compiler_params=_SC_PARAMS,
    )
    def k(ones, dst_i, out, dst_v, rows, acc, sem):
        c = lax.axis_index("c")
        s = lax.axis_index("s")
        pltpu.sync_copy(dst_i.at[s, pl.ds(c * CH_EDGE, CH_EDGE)], dst_v)
        pltpu.sync_copy(ones, rows)
        for j in range(rpt_deg // K):
            pltpu.sync_copy(rows, acc.at[pl.ds(s * rpt_deg + j * K, K)])
        plsc.subcore_barrier()

        def body(g, carry):
            for b in range(NBUF):
                i = g * NBUF + b
                pltpu.async_copy(rows, acc.at[dst_v.at[i]], sem[b], add=True)
            for b in range(NBUF):
                i = g * NBUF + b
                pltpu.make_async_copy(rows, acc.at[dst_v.at[i]],
                                      sem[b]).wait()
            return carry

        lax.fori_loop(0, NG, body, 0)
        plsc.subcore_barrier()
        base = s * RPT
        pltpu.sync_copy(acc.at[pl.ds(base, RPT)], out.at[c, pl.ds(base, RPT)])

        @pl.when(s == NS - 1)
        def _():
            pltpu.sync_copy(acc.at[pl.ds(NS * RPT, TAIL)],
                            out.at[c, pl.ds(NS * RPT, TAIL)])

    return k


def _dinv(d):
    return lax.rsqrt(d[0, :, :1] + d[1, :, :1] - 1.0)


def _row_spec(d):
    return pl.BlockSpec((BR, d), lambda i: (i, 0))


def _row3_spec(d):
    return pl.BlockSpec((NC, BR, d), lambda i: (0, i, 0))


def _full_spec(r, c):
    return pl.BlockSpec((r, c), lambda i: (0, 0))


def _mm1(x, w, degp):
    def body(x_r, w_r, d_r, o_r):
        h = jnp.dot(x_r[...], w_r[...], preferred_element_type=jnp.float32)
        o_r[...] = h * _dinv(d_r[...])

    return pl.pallas_call(
        body,
        grid=(GRID,),
        in_specs=[_row_spec(128), _full_spec(128, 64), _row3_spec(DEG_W)],
        out_specs=_row_spec(64),
        out_shape=jax.ShapeDtypeStruct((N, 64), jnp.float32),
    )(x, w, degp)


def _mm_mid(p, degp, w, b, din, dout, hs=None):
    pair = p.ndim == 3

    def body(*refs):
        if pair:
            p_r, hs_r, d_r, w_r, b_r, f_r, h_r = refs
            agg = p_r[0] + p_r[1] - hs_r[...]
        else:
            p_r, d_r, w_r, b_r, f_r, h_r = refs
            agg = p_r[...]
        dinv = _dinv(d_r[...])
        f = jnp.maximum(dinv * agg + b_r[...], 0.0)
        f_r[...] = f
        h_r[...] = jnp.dot(f, w_r[...], preferred_element_type=jnp.float32) * dinv

    if pair:
        args = [p, hs]
        specs = [_row3_spec(din), _row_spec(din)]
    else:
        args = [p]
        specs = [_row_spec(din)]
    return pl.pallas_call(
        body,
        grid=(GRID,),
        in_specs=specs + [_row3_spec(DEG_W),
                          _full_spec(din, dout), _full_spec(1, din)],
        out_specs=[_row_spec(din), _row_spec(dout)],
        out_shape=[jax.ShapeDtypeStruct((N, din), jnp.float32),
                   jax.ShapeDtypeStruct((N, dout), jnp.float32)],
    )(*args, degp, w, b)


def _mm_fc12(f1, f2, wf1, wf2, bfc):
    def body(f1_r, f2_r, w1_r, w2_r, bf_r, o_r):
        o_r[...] = (jnp.dot(f1_r[...], w1_r[...],
                            preferred_element_type=jnp.float32)
                    + jnp.dot(f2_r[...], w2_r[...],
                              preferred_element_type=jnp.float32)
                    + bf_r[...])

    return pl.pallas_call(
        body,
        grid=(GRID,),
        in_specs=[_row_spec(64), _row_spec(32),
                  _full_spec(64, 16), _full_spec(32, 16), _full_spec(1, 16)],
        out_specs=_row_spec(16),
        out_shape=jax.ShapeDtypeStruct((N, 16), jnp.float32),
    )(f1, f2, wf1, wf2, bfc)


def _mm_fin(p3, h3s, degp, fc12, b3, wf3):
    def body(p_r, hs_r, d_r, fc_r, b3_r, w3_r, o_r):
        dinv = _dinv(d_r[...])
        agg = p_r[0] + p_r[1] - hs_r[...]
        f3 = jnp.maximum(dinv * agg + b3_r[...], 0.0)
        acc = fc_r[...] + jnp.dot(f3, w3_r[...],
                                  preferred_element_type=jnp.float32)
        o_r[...] = jnp.maximum(acc, 0.0)

    return pl.pallas_call(
        body,
        grid=(GRID,),
        in_specs=[_row3_spec(16), _row_spec(16), _row3_spec(DEG_W),
                  _row_spec(16), _full_spec(1, 16), _full_spec(16, 16)],
        out_specs=_row_spec(16),
        out_shape=jax.ShapeDtypeStruct((N, 16), jnp.float32),
    )(p3, h3s, degp, fc12, b3, wf3)


def _pad_spread(idx, nper, npad, dst):
    if dst:
        pad = N + (jnp.arange(npad, dtype=jnp.int32) % (ACC_ROWS - N))
    else:
        pad = (jnp.arange(npad, dtype=jnp.int32) * 97) % N
    lead = idx.reshape(-1, nper)
    return jnp.concatenate(
        [lead, jnp.broadcast_to(pad, (lead.shape[0], npad))], axis=1)


def kernel(edges, features, W1, b1, W2, b2, W3, b3, Wfc, bfc):
    src = edges[0].astype(jnp.int32)
    dst = edges[1].astype(jnp.int32)
    ept = E // NS
    npad = CH_COL * K - ept
    src_c = _pad_spread(src, ept, npad, False).reshape(NS, CH_COL, K)
    dst_c = _pad_spread(dst, ept, npad, True).reshape(NS, CH_COL, K)
    ones = jnp.ones((K, DEG_W), jnp.float32)

    degp = _sc_degree()(ones, dst_c)

    h1s = _mm1(features, W1, degp)
    p1 = _sc_gather_scatter(64, split_cols=True)(h1s, src_c, dst_c)
    f1, h2s = _mm_mid(p1, degp, W2, b1.reshape(1, 64), 64, 32)
    p2 = _sc_gather_scatter(32, split_cols=False)(h2s, src_c, dst_c)
    f2, h3s = _mm_mid(p2, degp, W3, b2.reshape(1, 32), 32, 16, hs=h2s)
    p3 = _sc_gather_scatter(16, split_cols=False)(h3s, src_c, dst_c)
    fc12 = _mm_fc12(f1, f2, Wfc[:64], Wfc[64:96], bfc.reshape(1, 16))
    return _mm_fin(p3, h3s, degp, fc12, b3.reshape(1, 16), Wfc[96:])


# speedup vs baseline: 1.0446x; 1.0133x over previous
import functools

import jax
import jax.numpy as jnp
from jax import lax
from jax.experimental import pallas as pl
from jax.experimental.pallas import tpu as pltpu
from jax.experimental.pallas import tpu_sc as plsc

N = 10000
E = 320000
NC, NS = 2, 16
K = 128
ACC_ROWS = 10240
RPT = 624
TAIL = N - NS * RPT
NBUF = 10
DEG_W = 16

CH_COL = 160
CH_EDGE = 80

BR = 5000
GRID = N // BR

_SC_PARAMS = pltpu.CompilerParams(use_tc_tiling_on_sc=False)


def _sc_gather_scatter(D, split_cols):
    D2 = D // 2 if split_cols else D
    CH = CH_COL if split_cols else CH_EDGE
    nbuf = NBUF if split_cols else 16
    NG = CH // nbuf
    mesh = plsc.VectorSubcoreMesh(core_axis_name="c", subcore_axis_name="s")
    out_shape = (N, D) if split_cols else (NC, N, D)

    @functools.partial(
        pl.kernel,
        out_type=jax.ShapeDtypeStruct(out_shape, jnp.float32),
        mesh=mesh,
        scratch_types=[
            pltpu.VMEM((CH, K), jnp.int32),
            pltpu.VMEM((CH, K), jnp.int32),
            [pltpu.VMEM((K, D2), jnp.float32)] * nbuf,
            pltpu.VMEM_SHARED((N, D2), jnp.float32),
            pltpu.VMEM_SHARED((ACC_ROWS, D2), jnp.float32),
            [pltpu.SemaphoreType.DMA] * nbuf,
        ],
        compiler_params=_SC_PARAMS,
    )
    def k(hs, src_i, dst_i, out, src_v, dst_v, rows, hs_s, acc, sem):
        c = lax.axis_index("c")
        s = lax.axis_index("s")
        if split_cols:
            pltpu.sync_copy(src_i.at[s], src_v)
            pltpu.sync_copy(dst_i.at[s], dst_v)
        else:
            pltpu.sync_copy(src_i.at[s, pl.ds(c * CH, CH)], src_v)
            pltpu.sync_copy(dst_i.at[s, pl.ds(c * CH, CH)], dst_v)
        base = s * RPT
        co = c * D2 if split_cols else 0

        def stage(r0, nr):
            sl = (pl.ds(r0, nr), pl.ds(co, D2)) if split_cols else (pl.ds(r0, nr),)
            pltpu.sync_copy(hs.at[sl], hs_s.at[pl.ds(r0, nr)])
            pltpu.sync_copy(hs.at[sl], acc.at[pl.ds(r0, nr)])

        stage(base, RPT)

        @pl.when(s == NS - 1)
        def _():
            stage(NS * RPT, TAIL)

        plsc.subcore_barrier()

        for b in range(nbuf):
            pltpu.async_copy(hs_s.at[src_v.at[b]], rows[b], sem[b])

        def group(g, carry):
            for b in range(nbuf):
                i = g * nbuf + b
                pltpu.make_async_copy(hs_s.at[src_v.at[i]], rows[b],
                                      sem[b]).wait()
                pltpu.async_copy(rows[b], acc.at[dst_v.at[i]], sem[b],
                                 add=True)
            for b in range(nbuf):
                i = g * nbuf + b
                pltpu.make_async_copy(rows[b], acc.at[dst_v.at[i]],
                                      sem[b]).wait()

                @pl.when(g < NG - 1)
                def _():
                    j = (g + 1) * nbuf + b
                    pltpu.async_copy(hs_s.at[src_v.at[j]], rows[b], sem[b])

            return carry

        lax.fori_loop(0, NG, group, 0)
        plsc.subcore_barrier()

        def emit(r0, nr):
            sl = ((pl.ds(r0, nr), pl.ds(co, D2)) if split_cols
                  else (c, pl.ds(r0, nr)))
            pltpu.sync_copy(acc.at[pl.ds(r0, nr)], out.at[sl])

        emit(base, RPT)

        @pl.when(s == NS - 1)
        def _():
            emit(NS * RPT, TAIL)

    return k


def _sc_degree():
    mesh = plsc.VectorSubcoreMesh(core_axis_name="c", subcore_axis_name="s")
    rpt_deg = ACC_ROWS // NS
    NG = CH_EDGE // 16

    @functools.partial(
        pl.kernel,
        out_type=jax.ShapeDtypeStruct((NC, N, DEG_W), jnp.float32),
        mesh=mesh,
        scratch_types=[
            pltpu.VMEM((CH_EDGE, K), jnp.int32),
            pltpu.VMEM((K, DEG_W), jnp.float32),
            pltpu.VMEM_SHARED((ACC_ROWS, DEG_W), jnp.float32),
            [pltpu.SemaphoreType.DMA] * 16,
        ],
        compiler_params=_SC_PARAMS,
    )
    def k(ones, dst_i, out, dst_v, rows, acc, sem):
        c = lax.axis_index("c")
        s = lax.axis_index("s")
        pltpu.sync_copy(dst_i.at[s, pl.ds(c * CH_EDGE, CH_EDGE)], dst_v)
        pltpu.sync_copy(ones, rows)
        for j in range(rpt_deg // K):
            pltpu.sync_copy(rows, acc.at[pl.ds(s * rpt_deg + j * K, K)])
        plsc.subcore_barrier()

        def body(g, carry):
            for b in range(16):
                i = g * 16 + b
                pltpu.async_copy(rows, acc.at[dst_v.at[i]], sem[b], add=True)
            for b in range(16):
                i = g * 16 + b
                pltpu.make_async_copy(rows, acc.at[dst_v.at[i]],
                                      sem[b]).wait()
            return carry

        lax.fori_loop(0, NG, body, 0)
        plsc.subcore_barrier()
        base = s * RPT
        pltpu.sync_copy(acc.at[pl.ds(base, RPT)], out.at[c, pl.ds(base, RPT)])

        @pl.when(s == NS - 1)
        def _():
            pltpu.sync_copy(acc.at[pl.ds(NS * RPT, TAIL)],
                            out.at[c, pl.ds(NS * RPT, TAIL)])

    return k


def _dinv(d):
    return lax.rsqrt(d[0, :, :1] + d[1, :, :1] - 1.0)


def _row_spec(d):
    return pl.BlockSpec((BR, d), lambda i: (i, 0))


def _row3_spec(d):
    return pl.BlockSpec((NC, BR, d), lambda i: (0, i, 0))


def _full_spec(r, c):
    return pl.BlockSpec((r, c), lambda i: (0, 0))


def _mm1(x, w, degp):
    def body(x_r, w_r, d_r, o_r):
        h = jnp.dot(x_r[...], w_r[...], preferred_element_type=jnp.float32)
        o_r[...] = h * _dinv(d_r[...])

    return pl.pallas_call(
        body,
        grid=(GRID,),
        in_specs=[_row_spec(128), _full_spec(128, 64), _row3_spec(DEG_W)],
        out_specs=_row_spec(64),
        out_shape=jax.ShapeDtypeStruct((N, 64), jnp.float32),
    )(x, w, degp)


def _mm_mid(p, degp, w, b, din, dout, hs=None):
    pair = p.ndim == 3

    def body(*refs):
        if pair:
            p_r, hs_r, d_r, w_r, b_r, f_r, h_r = refs
            agg = p_r[0] + p_r[1] - hs_r[...]
        else:
            p_r, d_r, w_r, b_r, f_r, h_r = refs
            agg = p_r[...]
        dinv = _dinv(d_r[...])
        f = jnp.maximum(dinv * agg + b_r[...], 0.0)
        f_r[...] = f
        h_r[...] = jnp.dot(f, w_r[...], preferred_element_type=jnp.float32) * dinv

    if pair:
        args = [p, hs]
        specs = [_row3_spec(din), _row_spec(din)]
    else:
        args = [p]
        specs = [_row_spec(din)]
    return pl.pallas_call(
        body,
        grid=(GRID,),
        in_specs=specs + [_row3_spec(DEG_W),
                          _full_spec(din, dout), _full_spec(1, din)],
        out_specs=[_row_spec(din), _row_spec(dout)],
        out_shape=[jax.ShapeDtypeStruct((N, din), jnp.float32),
                   jax.ShapeDtypeStruct((N, dout), jnp.float32)],
    )(*args, degp, w, b)


def _mm_fc12(f1, f2, wf1, wf2, bfc):
    def body(f1_r, f2_r, w1_r, w2_r, bf_r, o_r):
        o_r[...] = (jnp.dot(f1_r[...], w1_r[...],
                            preferred_element_type=jnp.float32)
                    + jnp.dot(f2_r[...], w2_r[...],
                              preferred_element_type=jnp.float32)
                    + bf_r[...])

    return pl.pallas_call(
        body,
        grid=(GRID,),
        in_specs=[_row_spec(64), _row_spec(32),
                  _full_spec(64, 16), _full_spec(32, 16), _full_spec(1, 16)],
        out_specs=_row_spec(16),
        out_shape=jax.ShapeDtypeStruct((N, 16), jnp.float32),
    )(f1, f2, wf1, wf2, bfc)


def _mm_fin(p3, h3s, degp, fc12, b3, wf3):
    def body(p_r, hs_r, d_r, fc_r, b3_r, w3_r, o_r):
        dinv = _dinv(d_r[...])
        agg = p_r[0] + p_r[1] - hs_r[...]
        f3 = jnp.maximum(dinv * agg + b3_r[...], 0.0)
        acc = fc_r[...] + jnp.dot(f3, w3_r[...],
                                  preferred_element_type=jnp.float32)
        o_r[...] = jnp.maximum(acc, 0.0)

    return pl.pallas_call(
        body,
        grid=(GRID,),
        in_specs=[_row3_spec(16), _row_spec(16), _row3_spec(DEG_W),
                  _row_spec(16), _full_spec(1, 16), _full_spec(16, 16)],
        out_specs=_row_spec(16),
        out_shape=jax.ShapeDtypeStruct((N, 16), jnp.float32),
    )(p3, h3s, degp, fc12, b3, wf3)


def _pad_spread(idx, nper, npad, dst):
    if dst:
        pad = N + (jnp.arange(npad, dtype=jnp.int32) % (ACC_ROWS - N))
    else:
        pad = (jnp.arange(npad, dtype=jnp.int32) * 97) % N
    lead = idx.reshape(-1, nper)
    return jnp.concatenate(
        [lead, jnp.broadcast_to(pad, (lead.shape[0], npad))], axis=1)


def kernel(edges, features, W1, b1, W2, b2, W3, b3, Wfc, bfc):
    src = edges[0].astype(jnp.int32)
    dst = edges[1].astype(jnp.int32)
    ept = E // NS
    npad = CH_COL * K - ept
    src_c = _pad_spread(src, ept, npad, False).reshape(NS, CH_COL, K)
    dst_c = _pad_spread(dst, ept, npad, True).reshape(NS, CH_COL, K)
    ones = jnp.ones((K, DEG_W), jnp.float32)

    degp = _sc_degree()(ones, dst_c)

    h1s = _mm1(features, W1, degp)
    p1 = _sc_gather_scatter(64, split_cols=True)(h1s, src_c, dst_c)
    f1, h2s = _mm_mid(p1, degp, W2, b1.reshape(1, 64), 64, 32)
    p2 = _sc_gather_scatter(32, split_cols=False)(h2s, src_c, dst_c)
    f2, h3s = _mm_mid(p2, degp, W3, b2.reshape(1, 32), 32, 16, hs=h2s)
    p3 = _sc_gather_scatter(16, split_cols=False)(h3s, src_c, dst_c)
    fc12 = _mm_fc12(f1, f2, Wfc[:64], Wfc[64:96], bfc.reshape(1, 16))
    return _mm_fin(p3, h3s, degp, fc12, b3.reshape(1, 16), Wfc[96:])
